# baseline (speedup 1.0000x reference)
import functools

import jax
import jax.numpy as jnp
from jax import lax
from jax.experimental import pallas as pl
from jax.experimental.pallas import tpu as pltpu
from jax.experimental.pallas import tpu_sc as plsc

N = 10000
D = 128
G = 64
NC = 2
NS = 16
NW = NC * NS
NHALF = N // NC
HRPT = 320
HLAST = NHALF - (NS - 1) * HRPT
RPT = 640
LAST = N - (NS - 1) * RPT
EC = 80
DL = 16
RB = 400
NBLK = N // RB

_SC_MESH = plsc.VectorSubcoreMesh(core_axis_name="c", subcore_axis_name="s")


NCH = 250


EPW = 20000


@functools.partial(
    pl.kernel,
    mesh=_SC_MESH,
    out_type=jax.ShapeDtypeStruct((N, D), jnp.float32),
    scratch_types=[
        pltpu.VMEM((EPW,), jnp.int32),
        pltpu.VMEM((EPW,), jnp.int32),
        pltpu.VMEM((NCH, EC), jnp.int32),
        pltpu.VMEM((EC, D), jnp.float32),
        pltpu.VMEM_SHARED((NHALF + 8, D), jnp.float32),
        pltpu.SemaphoreType.DMA,
    ],
)
def _agg_kernel(hws_hbm, src_hbm, dst_hbm, acc_hbm,
                sidx_v, didx_v, ridx_v, rowsb_v, acc_sh, sem0):
    c = lax.axis_index("c")
    s = lax.axis_index("s")
    nbase = c * NHALF
    start = pl.multiple_of(s * HRPT, 8)
    rows = pl.ds(start, HRPT)
    rows_l = pl.ds(NHALF - HLAST, HLAST)
    gstart = pl.multiple_of(nbase + s * HRPT, 8)
    grows = pl.ds(gstart, HRPT)
    grows_l = pl.ds(pl.multiple_of(nbase + NHALF - HLAST, 8), HLAST)
    ebase = pl.multiple_of(s * EPW, 8)

    pltpu.sync_copy(src_hbm.at[pl.ds(ebase, EPW)], sidx_v)
    pltpu.sync_copy(dst_hbm.at[pl.ds(ebase, EPW)], didx_v)

    nfull = jnp.where(s < NS - 1, HRPT // EC, HLAST // EC)

    def _icp(k, carry):
        go = pl.multiple_of(gstart + k * EC, 8)
        lo = pl.multiple_of(start + k * EC, 8)
        pltpu.sync_copy(hws_hbm.at[pl.ds(go, EC)], rowsb_v)
        pltpu.sync_copy(rowsb_v, acc_sh.at[pl.ds(lo, EC)])
        return carry

    lax.fori_loop(0, nfull, _icp, 0)

    @pl.when(s == NS - 1)
    def _init_tail():
        tg = pl.multiple_of(gstart + (HLAST // EC) * EC, 8)
        tl = pl.multiple_of(start + (HLAST // EC) * EC, 8)
        tail = HLAST - (HLAST // EC) * EC
        pltpu.sync_copy(hws_hbm.at[pl.ds(tg, tail)],
                        rowsb_v.at[pl.ds(0, tail)])
        pltpu.sync_copy(rowsb_v.at[pl.ds(0, tail)],
                        acc_sh.at[pl.ds(tl, tail)])

    base16 = jnp.full((16,), nbase, jnp.int32)
    dump16 = jnp.full((16,), NHALF, jnp.int32)
    half16 = jnp.full((16,), NHALF, jnp.int32)

    def remap(i, carry):
        for j in range(EC // 16):
            v = didx_v[pl.ds(i * EC + j * 16, 16)] - base16
            ok = (v >= 0) & (v < half16)
            ridx_v[i, pl.ds(j * 16, 16)] = jnp.where(ok, v, dump16)
        return carry

    lax.fori_loop(0, NCH, remap, 0)
    plsc.subcore_barrier()

    def chunk(i, carry):
        pltpu.async_copy(
            hws_hbm.at[sidx_v.at[pl.ds(i * EC, EC)]],
            rowsb_v, sem0).wait()
        pltpu.sync_copy(rowsb_v,
                        acc_sh.at[ridx_v.at[i]], add=True)
        return carry

    lax.fori_loop(0, NCH, chunk, 0)
    plsc.subcore_barrier()

    def _ocp(k, carry):
        go = pl.multiple_of(gstart + k * EC, 8)
        lo = pl.multiple_of(start + k * EC, 8)
        pltpu.sync_copy(acc_sh.at[pl.ds(lo, EC)], rowsb_v)
        pltpu.sync_copy(rowsb_v, acc_hbm.at[pl.ds(go, EC)])
        return carry

    lax.fori_loop(0, nfull, _ocp, 0)

    @pl.when(s == NS - 1)
    def _out_tail():
        tg = pl.multiple_of(gstart + (HLAST // EC) * EC, 8)
        tl = pl.multiple_of(start + (HLAST // EC) * EC, 8)
        tail = HLAST - (HLAST // EC) * EC
        pltpu.sync_copy(acc_sh.at[pl.ds(tl, tail)],
                        rowsb_v.at[pl.ds(0, tail)])
        pltpu.sync_copy(rowsb_v.at[pl.ds(0, tail)],
                        acc_hbm.at[pl.ds(tg, tail)])


def _prep_body(a_ref, x_ref, w_ref, dinv_ref, hws_ref):
    dinv = lax.rsqrt(a_ref[...])
    hws_ref[...] = jnp.dot(
        x_ref[...], w_ref[...], preferred_element_type=jnp.float32) * dinv
    dinv_ref[...] = dinv


def _comb_body(a_ref, dinv_ref, b_ref, out_ref, st_ref):
    i = pl.program_id(0)
    o = dinv_ref[...] * a_ref[...] + b_ref[...][None, :]
    out_ref[...] = o

    @pl.when(i == 0)
    def _():
        st_ref[...] = jnp.zeros_like(st_ref)

    st_ref[0:1, :] += jnp.sum(o, axis=0, keepdims=True)
    st_ref[1:2, :] += jnp.sum(o * o, axis=0, keepdims=True)


def _nm_body(out_ref, st_ref, g_ref, be_ref, w_ref, dinv_ref, hws_ref):
    st = st_ref[...]
    m = st[0:1, :] * (1.0 / N)
    v = st[1:2, :] * (1.0 / N) - m * m
    rs = lax.rsqrt(v + 1e-5)
    h = jnp.maximum(
        (out_ref[...] - m) * rs * g_ref[...][None, :] + be_ref[...][None, :], 0.0)
    hws_ref[...] = jnp.dot(
        h, w_ref[...], preferred_element_type=jnp.float32) * dinv_ref[...]


def _fin_body(a_ref, dinv_ref, b_ref, h_ref):
    h_ref[...] = jnp.maximum(
        dinv_ref[...] * a_ref[...] + b_ref[...][None, :], 0.0)


def _pool_body(b_ref, h_ref, sum_ref, mx_ref, cnt_ref):
    i = pl.program_id(0)

    @pl.when(i == 0)
    def _():
        sum_ref[...] = jnp.zeros_like(sum_ref)
        cnt_ref[...] = jnp.zeros_like(cnt_ref)
        mx_ref[...] = jnp.full_like(mx_ref, -1e30)

    bvec = b_ref[0, 0, :]
    h = h_ref[...]
    oh = (bvec[:, None] == lax.broadcasted_iota(jnp.int32, (1, G), 1)
          ).astype(jnp.float32)
    sum_ref[...] += lax.dot_general(
        oh, h, (((0,), (0,)), ((), ())), preferred_element_type=jnp.float32)
    cnt_ref[...] += jnp.broadcast_to(jnp.sum(oh, axis=0)[:, None], (G, D))
    bmin = jnp.min(bvec)
    bmax = jnp.max(bvec)
    for g in range(G):
        @pl.when((bmin <= g) & (g <= bmax))
        def _upd():
            vals = jnp.where(bvec[:, None] == g, h, -1e30)
            mx_ref[g:g + 1, :] = jnp.maximum(
                mx_ref[g:g + 1, :], jnp.max(vals, axis=0, keepdims=True))


def _head_body(sum_ref, mx_ref, cnt_ref, lw1_ref, lb1_ref, lw2_ref, lb2_ref,
               out_ref):
    cnt = cnt_ref[...]
    mean = sum_ref[...] / jnp.maximum(cnt, 1.0)
    mxz = jnp.where(cnt > 0, mx_ref[...], 0.0)
    t = (jnp.dot(mean, lw1_ref[0:D, :], preferred_element_type=jnp.float32)
         + jnp.dot(mxz, lw1_ref[D:2 * D, :], preferred_element_type=jnp.float32)
         + lb1_ref[...][None, :])
    t = jnp.maximum(t, 0.0)
    out_ref[...] = jnp.dot(
        t, lw2_ref[...], preferred_element_type=jnp.float32) + lb2_ref[...][None, :]


_ROWB = pl.BlockSpec((RB, D), lambda i: (i, 0))
_WB = pl.BlockSpec((D, D), lambda i: (0, 0))
_VECB = pl.BlockSpec((D,), lambda i: (0,))
_STB = pl.BlockSpec((8, D), lambda i: (0, 0))
_ND_F32 = jax.ShapeDtypeStruct((N, D), jnp.float32)


def _prep(a, x, w1):
    return pl.pallas_call(
        _prep_body,
        grid=(NBLK,),
        in_specs=[_ROWB, _ROWB, _WB],
        out_specs=[_ROWB, _ROWB],
        out_shape=[_ND_F32, _ND_F32],
    )(a, x, w1)


def _comb(a, dinv, b):
    return pl.pallas_call(
        _comb_body,
        grid=(NBLK,),
        in_specs=[_ROWB, _ROWB, _VECB],
        out_specs=[_ROWB, _STB],
        out_shape=[_ND_F32, jax.ShapeDtypeStruct((8, D), jnp.float32)],
    )(a, dinv, b)


def _norm_mm(out, st, gg, be, w, dinv):
    return pl.pallas_call(
        _nm_body,
        grid=(NBLK,),
        in_specs=[_ROWB, _STB, _VECB, _VECB, _WB, _ROWB],
        out_specs=_ROWB,
        out_shape=_ND_F32,
    )(out, st, gg, be, w, dinv)


def _final(a, dinv, b):
    return pl.pallas_call(
        _fin_body,
        grid=(NBLK,),
        in_specs=[_ROWB, _ROWB, _VECB],
        out_specs=_ROWB,
        out_shape=_ND_F32,
    )(a, dinv, b)


def _pool(batch3d, h):
    gb = pl.BlockSpec((G, D), lambda i: (0, 0))
    return pl.pallas_call(
        _pool_body,
        grid=(NBLK,),
        in_specs=[pl.BlockSpec((1, 1, RB), lambda i: (i, 0, 0)), _ROWB],
        out_specs=[gb, gb, gb],
        out_shape=[jax.ShapeDtypeStruct((G, D), jnp.float32)] * 3,
    )(batch3d, h)


def _head(sums, mx, cnt, lw1, lb1, lw2p, lb2p):
    gb = pl.BlockSpec((G, D), lambda i: (0, 0))
    return pl.pallas_call(
        _head_body,
        grid=(1,),
        in_specs=[gb, gb, gb, pl.BlockSpec((2 * D, D), lambda i: (0, 0)),
                  _VECB, _WB, _VECB],
        out_specs=gb,
        out_shape=jax.ShapeDtypeStruct((G, D), jnp.float32),
    )(sums, mx, cnt, lw1, lb1, lw2p, lb2p)


def kernel(x, edge_index, batch, W1, b1, W2, b2, W3, b3, W4, b4,
           g1, be1, g2, be2, g3, be3, lw1, lb1, lw2, lb2):
    src = edge_index[0]
    dst = edge_index[1]
    batch3d = batch.astype(jnp.int32).reshape(NBLK, 1, RB)
    ncls = lw2.shape[1]
    lw2p = jnp.pad(lw2, ((0, 0), (0, D - ncls)))
    lb2p = jnp.pad(lb2, (0, D - ncls))

    ones_nd = jnp.ones((N, D), jnp.float32)

    wstack = jnp.stack([W1, W2, W3, W4, jnp.zeros_like(W4)])
    zv = jnp.zeros_like(b1)
    bstack = jnp.stack([zv, b1, b2, b3, b4])
    gstack = jnp.stack([zv, g1, g2, g3, zv])
    bestack = jnp.stack([zv, be1, be2, be3, zv])
    sel = jnp.array([0, 1, 1, 1, 2], jnp.int32)

    def _layer(carry, xs):
        hws, dv = carry
        w_l, b_l, g_l, be_l, sel_l = xs
        acc = _agg_kernel(hws, src, dst)

        def _br_prep(a):
            dinv2, hws2 = _prep(a, x, w_l)
            return (hws2, dinv2)

        def _br_mid(a):
            out, st = _comb(a, dv, b_l)
            return (_norm_mm(out, st, g_l, be_l, w_l, dv), dv)

        def _br_last(a):
            return (_final(a, dv, b_l), dv)

        return lax.switch(sel_l, (_br_prep, _br_mid, _br_last), acc), None

    (h4, _), _ = lax.scan(_layer, (ones_nd, ones_nd),
                          (wstack, bstack, gstack, bestack, sel))

    sums, mx, cnt = _pool(batch3d, h4)
    outp = _head(sums, mx, cnt, lw1, lb1, lw2p, lb2p)
    return outp[:, :ncls]


# speedup vs baseline: 1.3286x; 1.3286x over previous
import functools

import jax
import jax.numpy as jnp
from jax import lax
from jax.experimental import pallas as pl
from jax.experimental.pallas import tpu as pltpu
from jax.experimental.pallas import tpu_sc as plsc

N = 10000
D = 128
G = 64
NC = 2
NS = 16
NW = NC * NS
NHALF = N // NC
HRPT = 320
HLAST = NHALF - (NS - 1) * HRPT
RPT = 640
LAST = N - (NS - 1) * RPT
EC = 80
DL = 16
RB = 400
NBLK = N // RB

_SC_MESH = plsc.VectorSubcoreMesh(core_axis_name="c", subcore_axis_name="s")


NCH = 250


EPW = 20000


@functools.partial(
    pl.kernel,
    mesh=_SC_MESH,
    out_type=jax.ShapeDtypeStruct((N, D), jnp.float32),
    scratch_types=[
        pltpu.VMEM((EPW,), jnp.int32),
        pltpu.VMEM((NCH, EC), jnp.int32),
        pltpu.VMEM((2 * EC, D), jnp.float32),
        pltpu.VMEM_SHARED((NHALF + 8, D), jnp.float32),
        pltpu.SemaphoreType.DMA,
    ],
)
def _agg_kernel(hws_hbm, src_hbm, dst3_hbm, acc_hbm,
                sidx_v, ridx_v, rowsb_v, acc_sh, sem0):
    c = lax.axis_index("c")
    s = lax.axis_index("s")
    nbase = c * NHALF
    start = pl.multiple_of(s * HRPT, 8)
    rows = pl.ds(start, HRPT)
    rows_l = pl.ds(NHALF - HLAST, HLAST)
    gstart = pl.multiple_of(nbase + s * HRPT, 8)
    grows = pl.ds(gstart, HRPT)
    grows_l = pl.ds(pl.multiple_of(nbase + NHALF - HLAST, 8), HLAST)
    ebase = pl.multiple_of(s * EPW, 8)

    pltpu.sync_copy(src_hbm.at[pl.ds(ebase, EPW)], sidx_v)
    pltpu.sync_copy(dst3_hbm.at[s], ridx_v)

    nfull = jnp.where(s < NS - 1, HRPT // EC, HLAST // EC)

    def _icp(k, carry):
        go = pl.multiple_of(gstart + k * EC, 8)
        lo = pl.multiple_of(start + k * EC, 8)
        pltpu.sync_copy(hws_hbm.at[pl.ds(go, EC)], rowsb_v.at[pl.ds(0, EC)])
        pltpu.sync_copy(rowsb_v.at[pl.ds(0, EC)], acc_sh.at[pl.ds(lo, EC)])
        return carry

    lax.fori_loop(0, nfull, _icp, 0)

    @pl.when(s == NS - 1)
    def _init_tail():
        tg = pl.multiple_of(gstart + (HLAST // EC) * EC, 8)
        tl = pl.multiple_of(start + (HLAST // EC) * EC, 8)
        tail = HLAST - (HLAST // EC) * EC
        pltpu.sync_copy(hws_hbm.at[pl.ds(tg, tail)],
                        rowsb_v.at[pl.ds(0, tail)])
        pltpu.sync_copy(rowsb_v.at[pl.ds(0, tail)],
                        acc_sh.at[pl.ds(tl, tail)])

    base16 = jnp.full((16,), nbase, jnp.int32)
    dump16 = jnp.full((16,), NHALF, jnp.int32)
    half16 = jnp.full((16,), NHALF, jnp.int32)

    def remap(i, carry):
        for j in range(EC // 16):
            v = ridx_v[i, pl.ds(j * 16, 16)] - base16
            ok = (v >= 0) & (v < half16)
            ridx_v[i, pl.ds(j * 16, 16)] = jnp.where(ok, v, dump16)
        return carry

    lax.fori_loop(0, NCH, remap, 0)
    plsc.subcore_barrier()

    buf0 = rowsb_v.at[pl.ds(0, EC)]
    buf1 = rowsb_v.at[pl.ds(EC, EC)]

    def _gather(i, buf):
        return pltpu.make_async_copy(
            hws_hbm.at[sidx_v.at[pl.ds(i * EC, EC)]], buf, sem0)

    def pair(p, carry):
        i0 = p * 2
        _gather(i0, buf0).start()
        _gather(i0 + 1, buf1).start()
        _gather(i0, buf0).wait()
        pltpu.sync_copy(buf0, acc_sh.at[ridx_v.at[i0]], add=True)
        _gather(i0 + 1, buf1).wait()
        pltpu.sync_copy(buf1, acc_sh.at[ridx_v.at[i0 + 1]], add=True)
        return carry

    lax.fori_loop(0, NCH // 2, pair, 0)
    plsc.subcore_barrier()

    def _ocp(k, carry):
        go = pl.multiple_of(gstart + k * EC, 8)
        lo = pl.multiple_of(start + k * EC, 8)
        pltpu.sync_copy(acc_sh.at[pl.ds(lo, EC)], rowsb_v.at[pl.ds(0, EC)])
        pltpu.sync_copy(rowsb_v.at[pl.ds(0, EC)], acc_hbm.at[pl.ds(go, EC)])
        return carry

    lax.fori_loop(0, nfull, _ocp, 0)

    @pl.when(s == NS - 1)
    def _out_tail():
        tg = pl.multiple_of(gstart + (HLAST // EC) * EC, 8)
        tl = pl.multiple_of(start + (HLAST // EC) * EC, 8)
        tail = HLAST - (HLAST // EC) * EC
        pltpu.sync_copy(acc_sh.at[pl.ds(tl, tail)],
                        rowsb_v.at[pl.ds(0, tail)])
        pltpu.sync_copy(rowsb_v.at[pl.ds(0, tail)],
                        acc_hbm.at[pl.ds(tg, tail)])


def _prep_body(a_ref, x_ref, w_ref, dinv_ref, hws_ref):
    dinv = lax.rsqrt(a_ref[...])
    hws_ref[...] = jnp.dot(
        x_ref[...], w_ref[...], preferred_element_type=jnp.float32) * dinv
    dinv_ref[...] = dinv


def _comb_body(a_ref, dinv_ref, b_ref, out_ref, st_ref):
    i = pl.program_id(0)
    o = dinv_ref[...] * a_ref[...] + b_ref[...][None, :]
    out_ref[...] = o

    @pl.when(i == 0)
    def _():
        st_ref[...] = jnp.zeros_like(st_ref)

    st_ref[0:1, :] += jnp.sum(o, axis=0, keepdims=True)
    st_ref[1:2, :] += jnp.sum(o * o, axis=0, keepdims=True)


def _nm_body(out_ref, st_ref, g_ref, be_ref, w_ref, dinv_ref, hws_ref):
    st = st_ref[...]
    m = st[0:1, :] * (1.0 / N)
    v = st[1:2, :] * (1.0 / N) - m * m
    rs = lax.rsqrt(v + 1e-5)
    h = jnp.maximum(
        (out_ref[...] - m) * rs * g_ref[...][None, :] + be_ref[...][None, :], 0.0)
    hws_ref[...] = jnp.dot(
        h, w_ref[...], preferred_element_type=jnp.float32) * dinv_ref[...]


def _fin_body(a_ref, dinv_ref, b_ref, h_ref):
    h_ref[...] = jnp.maximum(
        dinv_ref[...] * a_ref[...] + b_ref[...][None, :], 0.0)


def _pool_body(b_ref, h_ref, sum_ref, mx_ref, cnt_ref):
    i = pl.program_id(0)

    @pl.when(i == 0)
    def _():
        sum_ref[...] = jnp.zeros_like(sum_ref)
        cnt_ref[...] = jnp.zeros_like(cnt_ref)
        mx_ref[...] = jnp.full_like(mx_ref, -1e30)

    bvec = b_ref[0, 0, :]
    h = h_ref[...]
    oh = (bvec[:, None] == lax.broadcasted_iota(jnp.int32, (1, G), 1)
          ).astype(jnp.float32)
    sum_ref[...] += lax.dot_general(
        oh, h, (((0,), (0,)), ((), ())), preferred_element_type=jnp.float32)
    cnt_ref[...] += jnp.broadcast_to(jnp.sum(oh, axis=0)[:, None], (G, D))
    bmin = jnp.min(bvec)
    bmax = jnp.max(bvec)
    for g in range(G):
        @pl.when((bmin <= g) & (g <= bmax))
        def _upd():
            vals = jnp.where(bvec[:, None] == g, h, -1e30)
            mx_ref[g:g + 1, :] = jnp.maximum(
                mx_ref[g:g + 1, :], jnp.max(vals, axis=0, keepdims=True))


def _head_body(sum_ref, mx_ref, cnt_ref, lw1_ref, lb1_ref, lw2_ref, lb2_ref,
               out_ref):
    cnt = cnt_ref[...]
    mean = sum_ref[...] / jnp.maximum(cnt, 1.0)
    mxz = jnp.where(cnt > 0, mx_ref[...], 0.0)
    t = (jnp.dot(mean, lw1_ref[0:D, :], preferred_element_type=jnp.float32)
         + jnp.dot(mxz, lw1_ref[D:2 * D, :], preferred_element_type=jnp.float32)
         + lb1_ref[...][None, :])
    t = jnp.maximum(t, 0.0)
    out_ref[...] = jnp.dot(
        t, lw2_ref[...], preferred_element_type=jnp.float32) + lb2_ref[...][None, :]


_ROWB = pl.BlockSpec((RB, D), lambda i: (i, 0))
_WB = pl.BlockSpec((D, D), lambda i: (0, 0))
_VECB = pl.BlockSpec((D,), lambda i: (0,))
_STB = pl.BlockSpec((8, D), lambda i: (0, 0))
_ND_F32 = jax.ShapeDtypeStruct((N, D), jnp.float32)


def _prep(a, x, w1):
    return pl.pallas_call(
        _prep_body,
        grid=(NBLK,),
        in_specs=[_ROWB, _ROWB, _WB],
        out_specs=[_ROWB, _ROWB],
        out_shape=[_ND_F32, _ND_F32],
    )(a, x, w1)


def _comb(a, dinv, b):
    return pl.pallas_call(
        _comb_body,
        grid=(NBLK,),
        in_specs=[_ROWB, _ROWB, _VECB],
        out_specs=[_ROWB, _STB],
        out_shape=[_ND_F32, jax.ShapeDtypeStruct((8, D), jnp.float32)],
    )(a, dinv, b)


def _norm_mm(out, st, gg, be, w, dinv):
    return pl.pallas_call(
        _nm_body,
        grid=(NBLK,),
        in_specs=[_ROWB, _STB, _VECB, _VECB, _WB, _ROWB],
        out_specs=_ROWB,
        out_shape=_ND_F32,
    )(out, st, gg, be, w, dinv)


def _final(a, dinv, b):
    return pl.pallas_call(
        _fin_body,
        grid=(NBLK,),
        in_specs=[_ROWB, _ROWB, _VECB],
        out_specs=_ROWB,
        out_shape=_ND_F32,
    )(a, dinv, b)


def _pool(batch3d, h):
    gb = pl.BlockSpec((G, D), lambda i: (0, 0))
    return pl.pallas_call(
        _pool_body,
        grid=(NBLK,),
        in_specs=[pl.BlockSpec((1, 1, RB), lambda i: (i, 0, 0)), _ROWB],
        out_specs=[gb, gb, gb],
        out_shape=[jax.ShapeDtypeStruct((G, D), jnp.float32)] * 3,
    )(batch3d, h)


def _head(sums, mx, cnt, lw1, lb1, lw2p, lb2p):
    gb = pl.BlockSpec((G, D), lambda i: (0, 0))
    return pl.pallas_call(
        _head_body,
        grid=(1,),
        in_specs=[gb, gb, gb, pl.BlockSpec((2 * D, D), lambda i: (0, 0)),
                  _VECB, _WB, _VECB],
        out_specs=gb,
        out_shape=jax.ShapeDtypeStruct((G, D), jnp.float32),
    )(sums, mx, cnt, lw1, lb1, lw2p, lb2p)


def kernel(x, edge_index, batch, W1, b1, W2, b2, W3, b3, W4, b4,
           g1, be1, g2, be2, g3, be3, lw1, lb1, lw2, lb2):
    src = edge_index[0]
    dst = edge_index[1]
    batch3d = batch.astype(jnp.int32).reshape(NBLK, 1, RB)
    ncls = lw2.shape[1]
    lw2p = jnp.pad(lw2, ((0, 0), (0, D - ncls)))
    lb2p = jnp.pad(lb2, (0, D - ncls))

    dst3 = dst.reshape(NS, NCH, EC)
    ones_nd = jnp.ones((N, D), jnp.float32)

    wstack = jnp.stack([W1, W2, W3, W4, jnp.zeros_like(W4)])
    zv = jnp.zeros_like(b1)
    bstack = jnp.stack([zv, b1, b2, b3, b4])
    gstack = jnp.stack([zv, g1, g2, g3, zv])
    bestack = jnp.stack([zv, be1, be2, be3, zv])
    sel = jnp.array([0, 1, 1, 1, 2], jnp.int32)

    def _layer(carry, xs):
        hws, dv = carry
        w_l, b_l, g_l, be_l, sel_l = xs
        acc = _agg_kernel(hws, src, dst3)

        def _br_prep(a):
            dinv2, hws2 = _prep(a, x, w_l)
            return (hws2, dinv2)

        def _br_mid(a):
            out, st = _comb(a, dv, b_l)
            return (_norm_mm(out, st, g_l, be_l, w_l, dv), dv)

        def _br_last(a):
            return (_final(a, dv, b_l), dv)

        return lax.switch(sel_l, (_br_prep, _br_mid, _br_last), acc), None

    (h4, _), _ = lax.scan(_layer, (ones_nd, ones_nd),
                          (wstack, bstack, gstack, bestack, sel))

    sums, mx, cnt = _pool(batch3d, h4)
    outp = _head(sums, mx, cnt, lw1, lb1, lw2p, lb2p)
    return outp[:, :ncls]


# speedup vs baseline: 1.4224x; 1.0706x over previous
import functools

import jax
import jax.numpy as jnp
from jax import lax
from jax.experimental import pallas as pl
from jax.experimental.pallas import tpu as pltpu
from jax.experimental.pallas import tpu_sc as plsc

N = 10000
D = 128
G = 64
NC = 2
NS = 16
NW = NC * NS
NHALF = N // NC
HRPT = 320
HLAST = NHALF - (NS - 1) * HRPT
RPT = 640
LAST = N - (NS - 1) * RPT
EC = 80
DL = 16
RB = 400
NBLK = N // RB

_SC_MESH = plsc.VectorSubcoreMesh(core_axis_name="c", subcore_axis_name="s")


NCH = 250


EPW = 20000


@functools.partial(
    pl.kernel,
    mesh=_SC_MESH,
    out_type=jax.ShapeDtypeStruct((N, D), jnp.float32),
    scratch_types=[
        pltpu.VMEM((EPW,), jnp.int32),
        pltpu.VMEM((NCH, EC), jnp.int32),
        pltpu.VMEM((3 * EC, D), jnp.float32),
        pltpu.VMEM_SHARED((NHALF + 8, D), jnp.float32),
        pltpu.SemaphoreType.DMA,
    ],
)
def _agg_kernel(hws_hbm, src_hbm, dst3_hbm, acc_hbm,
                sidx_v, ridx_v, rowsb_v, acc_sh, sem0):
    c = lax.axis_index("c")
    s = lax.axis_index("s")
    nbase = c * NHALF
    start = pl.multiple_of(s * HRPT, 8)
    rows = pl.ds(start, HRPT)
    rows_l = pl.ds(NHALF - HLAST, HLAST)
    gstart = pl.multiple_of(nbase + s * HRPT, 8)
    grows = pl.ds(gstart, HRPT)
    grows_l = pl.ds(pl.multiple_of(nbase + NHALF - HLAST, 8), HLAST)
    ebase = pl.multiple_of(s * EPW, 8)

    pltpu.sync_copy(src_hbm.at[pl.ds(ebase, EPW)], sidx_v)
    pltpu.sync_copy(dst3_hbm.at[s], ridx_v)

    nfull = jnp.where(s < NS - 1, HRPT // EC, HLAST // EC)

    def _icp(k, carry):
        go = pl.multiple_of(gstart + k * EC, 8)
        lo = pl.multiple_of(start + k * EC, 8)
        pltpu.sync_copy(hws_hbm.at[pl.ds(go, EC)], rowsb_v.at[pl.ds(0, EC)])
        pltpu.sync_copy(rowsb_v.at[pl.ds(0, EC)], acc_sh.at[pl.ds(lo, EC)])
        return carry

    lax.fori_loop(0, nfull, _icp, 0)

    @pl.when(s == NS - 1)
    def _init_tail():
        tg = pl.multiple_of(gstart + (HLAST // EC) * EC, 8)
        tl = pl.multiple_of(start + (HLAST // EC) * EC, 8)
        tail = HLAST - (HLAST // EC) * EC
        pltpu.sync_copy(hws_hbm.at[pl.ds(tg, tail)],
                        rowsb_v.at[pl.ds(0, tail)])
        pltpu.sync_copy(rowsb_v.at[pl.ds(0, tail)],
                        acc_sh.at[pl.ds(tl, tail)])

    base16 = jnp.full((16,), nbase, jnp.int32)
    dump16 = jnp.full((16,), NHALF, jnp.int32)
    half16 = jnp.full((16,), NHALF, jnp.int32)

    def remap(i, carry):
        for j in range(EC // 16):
            v = ridx_v[i, pl.ds(j * 16, 16)] - base16
            ok = (v >= 0) & (v < half16)
            ridx_v[i, pl.ds(j * 16, 16)] = jnp.where(ok, v, dump16)
        return carry

    lax.fori_loop(0, NCH, remap, 0)
    plsc.subcore_barrier()

    bufs = [rowsb_v.at[pl.ds(pl.multiple_of(k * EC, 8), EC)]
            for k in range(3)]

    def _gather(i, buf):
        return pltpu.make_async_copy(
            hws_hbm.at[sidx_v.at[pl.ds(i * EC, EC)]], buf, sem0)

    def tri(q, carry):
        i0 = q * 5
        for k in range(3):
            _gather(i0 + k, bufs[k]).start()
        for k in range(5):
            _gather(i0 + k, bufs[k % 3]).wait()
            pltpu.sync_copy(bufs[k % 3], acc_sh.at[ridx_v.at[i0 + k]],
                            add=True)
            if k + 3 < 5:
                _gather(i0 + k + 3, bufs[k % 3]).start()
        return carry

    lax.fori_loop(0, NCH // 5, tri, 0)
    plsc.subcore_barrier()

    def _ocp(k, carry):
        go = pl.multiple_of(gstart + k * EC, 8)
        lo = pl.multiple_of(start + k * EC, 8)
        pltpu.sync_copy(acc_sh.at[pl.ds(lo, EC)], rowsb_v.at[pl.ds(0, EC)])
        pltpu.sync_copy(rowsb_v.at[pl.ds(0, EC)], acc_hbm.at[pl.ds(go, EC)])
        return carry

    lax.fori_loop(0, nfull, _ocp, 0)

    @pl.when(s == NS - 1)
    def _out_tail():
        tg = pl.multiple_of(gstart + (HLAST // EC) * EC, 8)
        tl = pl.multiple_of(start + (HLAST // EC) * EC, 8)
        tail = HLAST - (HLAST // EC) * EC
        pltpu.sync_copy(acc_sh.at[pl.ds(tl, tail)],
                        rowsb_v.at[pl.ds(0, tail)])
        pltpu.sync_copy(rowsb_v.at[pl.ds(0, tail)],
                        acc_hbm.at[pl.ds(tg, tail)])


def _prep_body(a_ref, x_ref, w_ref, dinv_ref, hws_ref):
    dinv = lax.rsqrt(a_ref[...])
    hws_ref[...] = jnp.dot(
        x_ref[...], w_ref[...], preferred_element_type=jnp.float32) * dinv
    dinv_ref[...] = dinv


def _comb_body(a_ref, dinv_ref, b_ref, out_ref, st_ref):
    i = pl.program_id(0)
    o = dinv_ref[...] * a_ref[...] + b_ref[...][None, :]
    out_ref[...] = o

    @pl.when(i == 0)
    def _():
        st_ref[...] = jnp.zeros_like(st_ref)

    st_ref[0:1, :] += jnp.sum(o, axis=0, keepdims=True)
    st_ref[1:2, :] += jnp.sum(o * o, axis=0, keepdims=True)


def _nm_body(out_ref, st_ref, g_ref, be_ref, w_ref, dinv_ref, hws_ref):
    st = st_ref[...]
    m = st[0:1, :] * (1.0 / N)
    v = st[1:2, :] * (1.0 / N) - m * m
    rs = lax.rsqrt(v + 1e-5)
    h = jnp.maximum(
        (out_ref[...] - m) * rs * g_ref[...][None, :] + be_ref[...][None, :], 0.0)
    hws_ref[...] = jnp.dot(
        h, w_ref[...], preferred_element_type=jnp.float32) * dinv_ref[...]


def _fin_body(a_ref, dinv_ref, b_ref, h_ref):
    h_ref[...] = jnp.maximum(
        dinv_ref[...] * a_ref[...] + b_ref[...][None, :], 0.0)


def _pool_body(b_ref, h_ref, sum_ref, mx_ref, cnt_ref):
    i = pl.program_id(0)

    @pl.when(i == 0)
    def _():
        sum_ref[...] = jnp.zeros_like(sum_ref)
        cnt_ref[...] = jnp.zeros_like(cnt_ref)
        mx_ref[...] = jnp.full_like(mx_ref, -1e30)

    bvec = b_ref[0, 0, :]
    h = h_ref[...]
    oh = (bvec[:, None] == lax.broadcasted_iota(jnp.int32, (1, G), 1)
          ).astype(jnp.float32)
    sum_ref[...] += lax.dot_general(
        oh, h, (((0,), (0,)), ((), ())), preferred_element_type=jnp.float32)
    cnt_ref[...] += jnp.broadcast_to(jnp.sum(oh, axis=0)[:, None], (G, D))
    bmin = jnp.min(bvec)
    bmax = jnp.max(bvec)
    for g in range(G):
        @pl.when((bmin <= g) & (g <= bmax))
        def _upd():
            vals = jnp.where(bvec[:, None] == g, h, -1e30)
            mx_ref[g:g + 1, :] = jnp.maximum(
                mx_ref[g:g + 1, :], jnp.max(vals, axis=0, keepdims=True))


def _head_body(sum_ref, mx_ref, cnt_ref, lw1_ref, lb1_ref, lw2_ref, lb2_ref,
               out_ref):
    cnt = cnt_ref[...]
    mean = sum_ref[...] / jnp.maximum(cnt, 1.0)
    mxz = jnp.where(cnt > 0, mx_ref[...], 0.0)
    t = (jnp.dot(mean, lw1_ref[0:D, :], preferred_element_type=jnp.float32)
         + jnp.dot(mxz, lw1_ref[D:2 * D, :], preferred_element_type=jnp.float32)
         + lb1_ref[...][None, :])
    t = jnp.maximum(t, 0.0)
    out_ref[...] = jnp.dot(
        t, lw2_ref[...], preferred_element_type=jnp.float32) + lb2_ref[...][None, :]


_ROWB = pl.BlockSpec((RB, D), lambda i: (i, 0))
_WB = pl.BlockSpec((D, D), lambda i: (0, 0))
_VECB = pl.BlockSpec((D,), lambda i: (0,))
_STB = pl.BlockSpec((8, D), lambda i: (0, 0))
_ND_F32 = jax.ShapeDtypeStruct((N, D), jnp.float32)


def _prep(a, x, w1):
    return pl.pallas_call(
        _prep_body,
        grid=(NBLK,),
        in_specs=[_ROWB, _ROWB, _WB],
        out_specs=[_ROWB, _ROWB],
        out_shape=[_ND_F32, _ND_F32],
    )(a, x, w1)


def _comb(a, dinv, b):
    return pl.pallas_call(
        _comb_body,
        grid=(NBLK,),
        in_specs=[_ROWB, _ROWB, _VECB],
        out_specs=[_ROWB, _STB],
        out_shape=[_ND_F32, jax.ShapeDtypeStruct((8, D), jnp.float32)],
    )(a, dinv, b)


def _norm_mm(out, st, gg, be, w, dinv):
    return pl.pallas_call(
        _nm_body,
        grid=(NBLK,),
        in_specs=[_ROWB, _STB, _VECB, _VECB, _WB, _ROWB],
        out_specs=_ROWB,
        out_shape=_ND_F32,
    )(out, st, gg, be, w, dinv)


def _final(a, dinv, b):
    return pl.pallas_call(
        _fin_body,
        grid=(NBLK,),
        in_specs=[_ROWB, _ROWB, _VECB],
        out_specs=_ROWB,
        out_shape=_ND_F32,
    )(a, dinv, b)


def _pool(batch3d, h):
    gb = pl.BlockSpec((G, D), lambda i: (0, 0))
    return pl.pallas_call(
        _pool_body,
        grid=(NBLK,),
        in_specs=[pl.BlockSpec((1, 1, RB), lambda i: (i, 0, 0)), _ROWB],
        out_specs=[gb, gb, gb],
        out_shape=[jax.ShapeDtypeStruct((G, D), jnp.float32)] * 3,
    )(batch3d, h)


def _head(sums, mx, cnt, lw1, lb1, lw2p, lb2p):
    gb = pl.BlockSpec((G, D), lambda i: (0, 0))
    return pl.pallas_call(
        _head_body,
        grid=(1,),
        in_specs=[gb, gb, gb, pl.BlockSpec((2 * D, D), lambda i: (0, 0)),
                  _VECB, _WB, _VECB],
        out_specs=gb,
        out_shape=jax.ShapeDtypeStruct((G, D), jnp.float32),
    )(sums, mx, cnt, lw1, lb1, lw2p, lb2p)


def kernel(x, edge_index, batch, W1, b1, W2, b2, W3, b3, W4, b4,
           g1, be1, g2, be2, g3, be3, lw1, lb1, lw2, lb2):
    src = edge_index[0]
    dst = edge_index[1]
    batch3d = batch.astype(jnp.int32).reshape(NBLK, 1, RB)
    ncls = lw2.shape[1]
    lw2p = jnp.pad(lw2, ((0, 0), (0, D - ncls)))
    lb2p = jnp.pad(lb2, (0, D - ncls))

    dst3 = dst.reshape(NS, NCH, EC)
    ones_nd = jnp.ones((N, D), jnp.float32)

    wstack = jnp.stack([W1, W2, W3, W4, jnp.zeros_like(W4)])
    zv = jnp.zeros_like(b1)
    bstack = jnp.stack([zv, b1, b2, b3, b4])
    gstack = jnp.stack([zv, g1, g2, g3, zv])
    bestack = jnp.stack([zv, be1, be2, be3, zv])
    sel = jnp.array([0, 1, 1, 1, 2], jnp.int32)

    def _layer(carry, xs):
        hws, dv = carry
        w_l, b_l, g_l, be_l, sel_l = xs
        acc = _agg_kernel(hws, src, dst3)

        def _br_prep(a):
            dinv2, hws2 = _prep(a, x, w_l)
            return (hws2, dinv2)

        def _br_mid(a):
            out, st = _comb(a, dv, b_l)
            return (_norm_mm(out, st, g_l, be_l, w_l, dv), dv)

        def _br_last(a):
            return (_final(a, dv, b_l), dv)

        return lax.switch(sel_l, (_br_prep, _br_mid, _br_last), acc), None

    (h4, _), _ = lax.scan(_layer, (ones_nd, ones_nd),
                          (wstack, bstack, gstack, bestack, sel))

    sums, mx, cnt = _pool(batch3d, h4)
    outp = _head(sums, mx, cnt, lw1, lb1, lw2p, lb2p)
    return outp[:, :ncls]
